# Initial kernel scaffold; baseline (speedup 1.0000x reference)
import jax
import jax.numpy as jnp
from jax.experimental import pallas as pl


def kernel(x, edge_index, edge_weight, W, bias):
    raise NotImplementedError("write your pallas kernel here")


# speedup vs baseline: 3.7648x; 3.7648x over previous
import functools

import jax
import jax.numpy as jnp
from jax import lax
from jax.experimental import pallas as pl
from jax.experimental.pallas import tpu as pltpu
from jax.experimental.pallas import tpu_sc as plsc

N_NODES = 10000
N_EDGES = 160000
D_IN = 256
D_OUT = 256
DH = 128
NC = 2
NS = 16
CH = 128
NCH = 79
EPT = NCH * CH
E_PAD = NS * EPT
RPT = 624
RTAIL = N_NODES - NS * RPT

_MM_BLOCK = 1000

_BCAST_DNUMS = lax.GatherDimensionNumbers(
    offset_dims=(), collapsed_slice_dims=(0,), start_index_map=(0,)
)


def _lane_broadcast(vec16, j):
    return lax.gather(
        vec16,
        jnp.full((16, 1), j, jnp.int32),
        _BCAST_DNUMS,
        (1,),
        mode=lax.GatherScatterMode.PROMISE_IN_BOUNDS,
    )


def _matmul(x, W, bias2d):
    def body(x_ref, w_ref, b_ref, o_ref):
        o_ref[...] = (
            jnp.dot(x_ref[...], w_ref[...], preferred_element_type=jnp.float32)
            + b_ref[...]
        )

    return pl.pallas_call(
        body,
        grid=(N_NODES // _MM_BLOCK,),
        in_specs=[
            pl.BlockSpec((_MM_BLOCK, D_IN), lambda i: (i, 0)),
            pl.BlockSpec((D_IN, D_OUT), lambda i: (0, 0)),
            pl.BlockSpec((1, D_OUT), lambda i: (0, 0)),
        ],
        out_specs=pl.BlockSpec((_MM_BLOCK, D_OUT), lambda i: (i, 0)),
        out_shape=jax.ShapeDtypeStruct((N_NODES, D_OUT), jnp.float32),
    )(x, W, bias2d)


def _make_sc_agg():
    mesh = plsc.VectorSubcoreMesh(core_axis_name="c", subcore_axis_name="s")

    @functools.partial(
        pl.kernel,
        out_type=jax.ShapeDtypeStruct((NC, N_NODES, DH), jnp.float32),
        mesh=mesh,
        scratch_types=[
            pltpu.VMEM((NCH, CH), jnp.int32),
            pltpu.VMEM((NCH, CH), jnp.int32),
            pltpu.VMEM((CH, DH), jnp.float32),
            pltpu.VMEM((NCH, CH), jnp.float32),
            pltpu.VMEM_SHARED((N_NODES, DH), jnp.float32),
        ],
    )
    def agg(h2_hbm, srcsel_hbm, dst_hbm, w_hbm, out_hbm,
            src_v, dst_v, rows_v, w_v, acc_sh):
        c = lax.axis_index("c")
        s = lax.axis_index("s")

        @pl.loop(0, CH)
        def _(i):
            for r in range(DH // 16):
                rows_v[i, pl.ds(r * 16, 16)] = jnp.zeros((16,), jnp.float32)

        zbase = s * RPT
        for t in range(RPT // CH):
            pltpu.sync_copy(rows_v, acc_sh.at[pl.ds(zbase + t * CH, CH)])
        rem = RPT % CH
        if rem:
            pltpu.sync_copy(
                rows_v.at[pl.ds(0, rem)],
                acc_sh.at[pl.ds(zbase + (RPT // CH) * CH, rem)],
            )

        @pl.when(s == NS - 1)
        def _():
            pltpu.sync_copy(
                rows_v.at[pl.ds(0, RTAIL)],
                acc_sh.at[pl.ds(NS * RPT, RTAIL)],
            )

        plsc.subcore_barrier()

        pltpu.sync_copy(srcsel_hbm.at[c * NS + s], src_v)
        pltpu.sync_copy(dst_hbm.at[s], dst_v)
        pltpu.sync_copy(w_hbm.at[s], w_v)

        @pl.loop(0, NCH)
        def _(g):
            pltpu.sync_copy(h2_hbm.at[src_v.at[g]], rows_v)

            @pl.loop(0, CH // 16)
            def _(q):
                w16 = w_v[g, pl.ds(q * 16, 16)]
                for j in range(16):
                    wj = _lane_broadcast(w16, j)
                    e = q * 16 + j
                    for r in range(DH // 16):
                        sl = pl.ds(r * 16, 16)
                        rows_v[e, sl] = rows_v[e, sl] * wj

            pltpu.sync_copy(rows_v, acc_sh.at[dst_v.at[g]], add=True)

        plsc.subcore_barrier()
        pltpu.sync_copy(
            acc_sh.at[pl.ds(s * RPT, RPT)],
            out_hbm.at[c].at[pl.ds(s * RPT, RPT)],
        )

        @pl.when(s == NS - 1)
        def _():
            pltpu.sync_copy(
                acc_sh.at[pl.ds(NS * RPT, RTAIL)],
                out_hbm.at[c].at[pl.ds(NS * RPT, RTAIL)],
            )

    return agg


_sc_agg = _make_sc_agg()


def kernel(x, edge_index, edge_weight, W, bias):
    h = _matmul(x, W, bias.reshape(1, D_OUT))
    h2 = h.reshape(2 * N_NODES, DH)

    dst = edge_index[0].astype(jnp.int32)
    src = edge_index[1].astype(jnp.int32)
    w = edge_weight.astype(jnp.float32)

    pad = E_PAD - N_EDGES
    src = jnp.pad(src, (0, pad))
    dst = jnp.pad(dst, (0, pad))
    w = jnp.pad(w, (0, pad))

    srcsel = jnp.stack([2 * src, 2 * src + 1]).reshape(2 * NS, NCH, CH)
    dstr = dst.reshape(NS, NCH, CH)
    wr = w.reshape(NS, NCH, CH)

    out2 = _sc_agg(h2, srcsel, dstr, wr)
    return jnp.concatenate([out2[0], out2[1]], axis=1)
